# baseline (speedup 1.0000x reference)
import jax
import jax.numpy as jnp
from jax.experimental import pallas as pl
from jax.experimental.pallas import tpu as pltpu

N0 = 8192
N1 = 8192
D_EDGE = 128
D_OUT = 128

_BM = 128


def _h_kernel(x1_ref, wt_ref, b_ref, h_ref):
    x1 = x1_ref[...].astype(jnp.bfloat16)
    wt = wt_ref[...].astype(jnp.bfloat16)
    h = jnp.dot(x1, wt, preferred_element_type=jnp.float32) + b_ref[...]
    h_ref[...] = h.astype(jnp.bfloat16)


def _mm_kernel(pw_ref, d_ref, h_ref, o_ref):
    d = d_ref[...].astype(jnp.bfloat16)
    acc = jnp.dot(d, h_ref[...], preferred_element_type=jnp.float32)
    w = pw_ref[0]
    o_ref[...] = jnp.where(acc >= 0, acc, w * acc)


def kernel(X0, X1, X2, L0, L1, L2, B2D3, D2B1TD1inv, D1invB1, B2TD2inv, W_e2n, b_e2n, prelu_w):
    h = pl.pallas_call(
        _h_kernel,
        grid=(8,),
        in_specs=[
            pl.BlockSpec((N1 // 8, D_EDGE), lambda i: (i, 0)),
            pl.BlockSpec((D_EDGE, D_OUT), lambda i: (0, 0)),
            pl.BlockSpec((1, D_OUT), lambda i: (0, 0)),
        ],
        out_specs=pl.BlockSpec((N1 // 8, D_OUT), lambda i: (i, 0)),
        out_shape=jax.ShapeDtypeStruct((N1, D_OUT), jnp.bfloat16),
        compiler_params=pltpu.CompilerParams(
            dimension_semantics=("parallel",),
        ),
    )(X1, W_e2n.T, b_e2n.reshape(1, D_OUT))

    grid = (N0 // _BM,)
    y = pl.pallas_call(
        _mm_kernel,
        grid=grid,
        in_specs=[
            pl.BlockSpec(memory_space=pltpu.SMEM),
            pl.BlockSpec((_BM, N1), lambda i: (i, 0)),
            pl.BlockSpec((N1, D_OUT), lambda i: (0, 0)),
        ],
        out_specs=pl.BlockSpec((_BM, D_OUT), lambda i: (i, 0)),
        out_shape=jax.ShapeDtypeStruct((N0, D_OUT), jnp.float32),
        compiler_params=pltpu.CompilerParams(
            dimension_semantics=("arbitrary",),
        ),
    )(prelu_w, D1invB1, h)
    return y


# speedup vs baseline: 1.3123x; 1.3123x over previous
import jax
import jax.numpy as jnp
from jax.experimental import pallas as pl
from jax.experimental.pallas import tpu as pltpu

N0 = 8192
N1 = 8192
D_EDGE = 128
D_OUT = 128

_BM = 256


def _fused_kernel(pw_ref, d_ref, x1_ref, wt_ref, b_ref, o_ref, h_ref):
    i = pl.program_id(0)

    @pl.when(i == 0)
    def _():
        x1 = x1_ref[...].astype(jnp.bfloat16)
        wt = wt_ref[...].astype(jnp.bfloat16)
        h = jnp.dot(x1, wt, preferred_element_type=jnp.float32) + b_ref[...]
        h_ref[...] = h.astype(jnp.bfloat16)

    d = d_ref[...].astype(jnp.bfloat16)
    acc = jnp.dot(d, h_ref[...], preferred_element_type=jnp.float32)
    w = pw_ref[0]
    o_ref[...] = jnp.where(acc >= 0, acc, w * acc)


def kernel(X0, X1, X2, L0, L1, L2, B2D3, D2B1TD1inv, D1invB1, B2TD2inv, W_e2n, b_e2n, prelu_w):
    grid = (N0 // _BM,)
    y = pl.pallas_call(
        _fused_kernel,
        grid=grid,
        in_specs=[
            pl.BlockSpec(memory_space=pltpu.SMEM),
            pl.BlockSpec((_BM, N1), lambda i: (i, 0)),
            pl.BlockSpec((N1, D_EDGE), lambda i: (0, 0)),
            pl.BlockSpec((D_EDGE, D_OUT), lambda i: (0, 0)),
            pl.BlockSpec((1, D_OUT), lambda i: (0, 0)),
        ],
        out_specs=pl.BlockSpec((_BM, D_OUT), lambda i: (i, 0)),
        out_shape=jax.ShapeDtypeStruct((N0, D_OUT), jnp.float32),
        scratch_shapes=[pltpu.VMEM((N1, D_OUT), jnp.bfloat16)],
        compiler_params=pltpu.CompilerParams(
            dimension_semantics=("arbitrary",),
        ),
    )(prelu_w, D1invB1, X1, W_e2n.T, b_e2n.reshape(1, D_OUT))
    return y
